# Initial kernel scaffold; baseline (speedup 1.0000x reference)
import jax
import jax.numpy as jnp
from jax.experimental import pallas as pl


def kernel(x, Wq1, bq1, Wk1, bk1, Wv1, bv1, We1, Wsk1, bsk1, gn1w, gn1b, gn1ms, Wq2, bq2, Wk2, bk2, Wv2, bv2, Wsk2, bsk2, gn2w, gn2b, gn2ms):
    raise NotImplementedError("write your pallas kernel here")


# speedup vs baseline: 1844.7512x; 1844.7512x over previous
import math

import jax
import jax.numpy as jnp
from jax.experimental import pallas as pl

LR = 160
HR = 320
F1 = 2 * HR
H1 = 8
C1 = F1 // H1
F2 = HR
H2 = 4
C2 = F2 // H2

_DN11 = (((1,), (1,)), ((), ()))
_DN00 = (((0,), (0,)), ((), ()))


def _dot(a, b):
    return jnp.dot(a, b, preferred_element_type=jnp.float32)


def _attention(q, k, v, n_heads, c, xt=None, we=None):
    inv = 1.0 / math.sqrt(c)
    outs = []
    for h in range(n_heads):
        sl = slice(h * c, (h + 1) * c)
        qh, kh, vh = q[:, sl], k[:, sl], v[:, sl]
        logits = jax.lax.dot_general(qh, kh, _DN11,
                                     preferred_element_type=jnp.float32)
        if xt is not None:
            weh = we[:, sl]
            uh = jnp.sum(qh * weh, axis=1, keepdims=True)
            logits = logits + uh * xt
        logits = logits * inv
        m = jnp.max(logits, axis=1, keepdims=True)
        p = jnp.exp(logits - m)
        p = p / (jnp.sum(p, axis=1, keepdims=True) + 1e-16)
        aggh = _dot(p, vh)
        if xt is not None:
            aggh = aggh + jnp.sum(p * xt, axis=1, keepdims=True) * weh
        outs.append(aggh)
    return jnp.concatenate(outs, axis=1)


def _graphnorm(x, w, b, ms):
    mean = jnp.mean(x, axis=0, keepdims=True)
    out = x - mean * ms
    var = jnp.mean(out * out, axis=0, keepdims=True)
    return w * out / jnp.sqrt(var + 1e-5) + b


def _body(x_ref, wq1_ref, bq1_ref, wk1_ref, bk1_ref, wv1_ref, bv1_ref,
          we1_ref, wsk1_ref, bsk1_ref, gn1w_ref, gn1b_ref, gn1ms_ref,
          wq2_ref, bq2_ref, wk2_ref, bk2_ref, wv2_ref, bv2_ref,
          wsk2_ref, bsk2_ref, gn2w_ref, gn2b_ref, gn2ms_ref,
          hr_ref, lr_ref):
    x = x_ref[...]

    q1 = _dot(x, wq1_ref[...]) + bq1_ref[...]
    k1 = _dot(x, wk1_ref[...]) + bk1_ref[...]
    v1 = _dot(x, wv1_ref[...]) + bv1_ref[...]
    xt = x.T
    h1 = _attention(q1, k1, v1, H1, C1, xt=xt, we=we1_ref[...])
    h1 = h1 + _dot(x, wsk1_ref[...]) + bsk1_ref[...]
    h1 = _graphnorm(h1, gn1w_ref[...], gn1b_ref[...], gn1ms_ref[...])
    lr_x = h1 / jnp.sqrt(jnp.sum(h1 * h1, axis=1, keepdims=True))
    lr_ref[...] = jnp.maximum(
        jax.lax.dot_general(lr_x, lr_x, _DN11,
                            preferred_element_type=jnp.float32), 0.0)

    q2 = jax.lax.dot_general(lr_x, wq2_ref[...], _DN00,
                             preferred_element_type=jnp.float32) + bq2_ref[...]
    k2 = jax.lax.dot_general(lr_x, wk2_ref[...], _DN00,
                             preferred_element_type=jnp.float32) + bk2_ref[...]
    v2 = jax.lax.dot_general(lr_x, wv2_ref[...], _DN00,
                             preferred_element_type=jnp.float32) + bv2_ref[...]
    h2 = _attention(q2, k2, v2, H2, C2)
    h2 = h2 + jax.lax.dot_general(lr_x, wsk2_ref[...], _DN00,
                                  preferred_element_type=jnp.float32) \
            + bsk2_ref[...]
    g = _graphnorm(h2, gn2w_ref[...], gn2b_ref[...], gn2ms_ref[...])
    gg = g / jnp.sqrt(jnp.sum(g * g, axis=0, keepdims=True))
    hr_ref[...] = jnp.maximum(
        jax.lax.dot_general(gg, gg, _DN00,
                            preferred_element_type=jnp.float32), 0.0)


def kernel(x, Wq1, bq1, Wk1, bk1, Wv1, bv1, We1, Wsk1, bsk1, gn1w, gn1b,
           gn1ms, Wq2, bq2, Wk2, bk2, Wv2, bv2, Wsk2, bsk2, gn2w, gn2b,
           gn2ms):
    row = lambda a: a.reshape(1, -1)
    return pl.pallas_call(
        _body,
        out_shape=(
            jax.ShapeDtypeStruct((HR, HR), jnp.float32),
            jax.ShapeDtypeStruct((LR, LR), jnp.float32),
        ),
    )(x, Wq1, row(bq1), Wk1, row(bk1), Wv1, row(bv1), We1, Wsk1, row(bsk1),
      row(gn1w), row(gn1b), row(gn1ms), Wq2, row(bq2), Wk2, row(bk2), Wv2,
      row(bv2), Wsk2, row(bsk2), row(gn2w), row(gn2b), row(gn2ms))


# speedup vs baseline: 1935.9098x; 1.0494x over previous
import math

import jax
import jax.numpy as jnp
from jax.experimental import pallas as pl

LR = 160
HR = 320
F1 = 2 * HR
H1 = 8
C1 = F1 // H1
F2 = HR
H2 = 4
C2 = F2 // H2

_DN11 = (((1,), (1,)), ((), ()))
_DN00 = (((0,), (0,)), ((), ()))


def _dot(a, b):
    return jnp.dot(a, b, preferred_element_type=jnp.float32)


def _attention(q, k, v, n_heads, c, xt=None, we=None):
    inv = 1.0 / math.sqrt(c)
    outs = []
    for h in range(n_heads):
        sl = slice(h * c, (h + 1) * c)
        qh, kh, vh = q[:, sl] * inv, k[:, sl], v[:, sl]
        logits = jax.lax.dot_general(qh, kh, _DN11,
                                     preferred_element_type=jnp.float32)
        if xt is not None:
            weh = we[:, sl]
            uh = jnp.sum(qh * weh, axis=1, keepdims=True)
            logits = logits + uh * xt
        m = jnp.max(logits, axis=1, keepdims=True)
        p = jnp.exp(logits - m)
        den = jnp.sum(p, axis=1, keepdims=True) + 1e-16
        aggh = _dot(p, vh)
        if xt is not None:
            aggh = aggh + jnp.sum(p * xt, axis=1, keepdims=True) * weh
        outs.append(aggh / den)
    return jnp.concatenate(outs, axis=1)


def _graphnorm(x, w, b, ms):
    mean = jnp.mean(x, axis=0, keepdims=True)
    out = x - mean * ms
    var = jnp.mean(out * out, axis=0, keepdims=True)
    return w * out / jnp.sqrt(var + 1e-5) + b


def _body(x_ref, wq1_ref, bq1_ref, wk1_ref, bk1_ref, wv1_ref, bv1_ref,
          we1_ref, wsk1_ref, bsk1_ref, gn1w_ref, gn1b_ref, gn1ms_ref,
          wq2_ref, bq2_ref, wk2_ref, bk2_ref, wv2_ref, bv2_ref,
          wsk2_ref, bsk2_ref, gn2w_ref, gn2b_ref, gn2ms_ref,
          hr_ref, lr_ref):
    x = x_ref[...]

    q1 = _dot(x, wq1_ref[...]) + bq1_ref[...]
    k1 = _dot(x, wk1_ref[...]) + bk1_ref[...]
    v1 = _dot(x, wv1_ref[...]) + bv1_ref[...]
    xt = x.T
    h1 = _attention(q1, k1, v1, H1, C1, xt=xt, we=we1_ref[...])
    h1 = h1 + _dot(x, wsk1_ref[...]) + bsk1_ref[...]
    h1 = _graphnorm(h1, gn1w_ref[...], gn1b_ref[...], gn1ms_ref[...])
    lr_x = h1 / jnp.sqrt(jnp.sum(h1 * h1, axis=1, keepdims=True))
    lr_ref[...] = jnp.maximum(
        jax.lax.dot_general(lr_x, lr_x, _DN11,
                            preferred_element_type=jnp.float32), 0.0)

    q2 = jax.lax.dot_general(lr_x, wq2_ref[...], _DN00,
                             preferred_element_type=jnp.float32) + bq2_ref[...]
    k2 = jax.lax.dot_general(lr_x, wk2_ref[...], _DN00,
                             preferred_element_type=jnp.float32) + bk2_ref[...]
    v2 = jax.lax.dot_general(lr_x, wv2_ref[...], _DN00,
                             preferred_element_type=jnp.float32) + bv2_ref[...]
    h2 = _attention(q2, k2, v2, H2, C2)
    h2 = h2 + jax.lax.dot_general(lr_x, wsk2_ref[...], _DN00,
                                  preferred_element_type=jnp.float32) \
            + bsk2_ref[...]
    g = _graphnorm(h2, gn2w_ref[...], gn2b_ref[...], gn2ms_ref[...])
    gg = g / jnp.sqrt(jnp.sum(g * g, axis=0, keepdims=True))
    hr_ref[...] = jnp.maximum(
        jax.lax.dot_general(gg, gg, _DN00,
                            preferred_element_type=jnp.float32), 0.0)


def kernel(x, Wq1, bq1, Wk1, bk1, Wv1, bv1, We1, Wsk1, bsk1, gn1w, gn1b,
           gn1ms, Wq2, bq2, Wk2, bk2, Wv2, bv2, Wsk2, bsk2, gn2w, gn2b,
           gn2ms):
    row = lambda a: a.reshape(1, -1)
    return pl.pallas_call(
        _body,
        out_shape=(
            jax.ShapeDtypeStruct((HR, HR), jnp.float32),
            jax.ShapeDtypeStruct((LR, LR), jnp.float32),
        ),
    )(x, Wq1, row(bq1), Wk1, row(bk1), Wv1, row(bv1), We1, Wsk1, row(bsk1),
      row(gn1w), row(gn1b), row(gn1ms), Wq2, row(bq2), Wk2, row(bk2), Wv2,
      row(bv2), Wsk2, row(bsk2), row(gn2w), row(gn2b), row(gn2ms))
